# baseline (device time: 55977 ns/iter reference)
import jax
import jax.numpy as jnp
from jax import lax
from jax.experimental import pallas as pl
from jax.experimental.pallas import tpu as pltpu

N_DEV = 32
B, SQ, D_MODEL, HQ, DH = 2, 512, 768, 8, 64
DQK = HQ * DH
SKV_LOC = 512
COLS = DQK + HQ
CHUNK = SQ // N_DEV

_MESH = pl.DeviceIdType.MESH


def _regroup(a):
    return jnp.swapaxes(a.reshape(2, 4, 64, DH), 0, 1).reshape(4, 128, DH)


def _ungroup(a, n=DH):
    return jnp.swapaxes(a.reshape(4, 2, 64, n), 0, 1).reshape(SQ, n)


def kernel(x, Wq, K_ext, V_ext, Wo):
    def body(x_ref, wq_ref, k_ref, v_ref, wo_ref, out_ref,
             acc_ref, sendbuf_ref, stage_ref, ostage_ref,
             send1, recv1, send2, recv2):
        d = lax.axis_index("i")

        barrier_sem = pltpu.get_barrier_semaphore()
        for o in range(1, N_DEV):
            t = lax.rem(d + o, N_DEV)
            pl.semaphore_signal(barrier_sem, inc=1, device_id=(t,),
                                device_id_type=_MESH)
        pl.semaphore_wait(barrier_sem, N_DEV - 1)

        r1 = [[], []]
        for b in range(B):
            qm = jnp.dot(x_ref[b], wq_ref[...],
                         preferred_element_type=jnp.float32)
            l_cols = []
            for h in range(HQ):
                qg = _regroup(qm[:, h * DH:(h + 1) * DH])
                kg = _regroup(k_ref[b, :, h, :])
                vg = _regroup(v_ref[b, :, h, :])
                sc = lax.dot_general(
                    qg, kg, (((2,), (2,)), ((0,), (0,))),
                    preferred_element_type=jnp.float32) * 0.125
                w = jnp.exp(sc)
                l_cols.append(
                    _ungroup(jnp.sum(w, axis=2, keepdims=True), n=1))
                og = lax.dot_general(
                    w, vg, (((2,), (1,)), ((0,), (0,))),
                    preferred_element_type=jnp.float32)
                acc_ref[b * SQ:(b + 1) * SQ,
                        h * DH:(h + 1) * DH] = _ungroup(og)
            acc_ref[b * SQ:(b + 1) * SQ, DQK:] = jnp.concatenate(
                l_cols, axis=1)
            sendbuf_ref[b * SQ:(b + 1) * SQ, :] = acc_ref[
                b * SQ:(b + 1) * SQ, :].astype(jnp.bfloat16)

            for o in range(1, N_DEV):
                t = lax.rem(d + o, N_DEV)
                r = pltpu.make_async_remote_copy(
                    src_ref=sendbuf_ref.at[pl.ds(b * SQ + t * CHUNK, CHUNK), :],
                    dst_ref=stage_ref.at[b, o],
                    send_sem=send1.at[b, o],
                    recv_sem=recv1.at[b, o],
                    device_id=(t,),
                    device_id_type=_MESH,
                )
                r.start()
                r1[b].append(r)

        r2 = []
        for b in range(B):
            for r in r1[b]:
                r.wait()
            rows = pl.ds(b * SQ + d * CHUNK, CHUNK)
            red = acc_ref[rows, :] + jnp.sum(
                stage_ref[b, 1:, :, :].astype(jnp.float32), axis=0)
            ctx = jnp.concatenate(
                [red[:, h * DH:(h + 1) * DH] / red[:, DQK + h:DQK + h + 1]
                 for h in range(HQ)], axis=1)
            outc = jnp.dot(ctx, wo_ref[...],
                           preferred_element_type=jnp.float32)
            orows = pl.ds(d * CHUNK, CHUNK)
            ostage_ref[b, orows, :] = outc.astype(jnp.bfloat16)
            for o in range(1, N_DEV):
                t = lax.rem(d + o, N_DEV)
                r = pltpu.make_async_remote_copy(
                    src_ref=ostage_ref.at[b, orows, :],
                    dst_ref=ostage_ref.at[b, orows, :],
                    send_sem=send2.at[b, o],
                    recv_sem=recv2.at[b, o],
                    device_id=(t,),
                    device_id_type=_MESH,
                )
                r.start()
                r2.append(r)
        for r in r2:
            r.wait()
        out_ref[...] = ostage_ref[...].astype(jnp.float32)

    return pl.pallas_call(
        body,
        out_shape=jax.ShapeDtypeStruct((B, SQ, D_MODEL), jnp.float32),
        in_specs=[pl.BlockSpec(memory_space=pltpu.VMEM)] * 5,
        out_specs=pl.BlockSpec(memory_space=pltpu.VMEM),
        scratch_shapes=[
            pltpu.VMEM((B * SQ, COLS), jnp.float32),
            pltpu.VMEM((B * SQ, COLS), jnp.bfloat16),
            pltpu.VMEM((B, N_DEV, CHUNK, COLS), jnp.bfloat16),
            pltpu.VMEM((B, SQ, D_MODEL), jnp.bfloat16),
            pltpu.SemaphoreType.DMA((B, N_DEV)),
            pltpu.SemaphoreType.DMA((B, N_DEV)),
            pltpu.SemaphoreType.DMA((B, N_DEV)),
            pltpu.SemaphoreType.DMA((B, N_DEV)),
        ],
        compiler_params=pltpu.CompilerParams(collective_id=0),
    )(x, Wq, K_ext, V_ext, Wo)


# device time: 52260 ns/iter; 1.0711x vs baseline; 1.0711x over previous
import jax
import jax.numpy as jnp
from jax import lax
from jax.experimental import pallas as pl
from jax.experimental.pallas import tpu as pltpu

N_DEV = 32
B, SQ, D_MODEL, HQ, DH = 2, 512, 768, 8, 64
DQK = HQ * DH
SKV_LOC = 512
COLS = DQK + HQ
CHUNK = SQ // N_DEV

_MESH = pl.DeviceIdType.MESH


def _regroup(a):
    return jnp.swapaxes(a.reshape(2, 4, 64, DH), 0, 1).reshape(4, 128, DH)


def _ungroup(a, n=DH):
    return jnp.swapaxes(a.reshape(4, 2, 64, n), 0, 1).reshape(SQ, n)


def kernel(x, Wq, K_ext, V_ext, Wo):
    def body(x_ref, wq_ref, k_ref, v_ref, wo_ref, out_ref,
             acc_ref, sendbuf_ref, stage_ref, ostage_ref,
             send1, recv1, send2, recv2):
        d = lax.axis_index("i")

        barrier_sem = pltpu.get_barrier_semaphore()
        for o in range(1, N_DEV):
            t = lax.rem(d + o, N_DEV)
            pl.semaphore_signal(barrier_sem, inc=1, device_id=(t,),
                                device_id_type=_MESH)
        pl.semaphore_wait(barrier_sem, N_DEV - 1)

        wqb = wq_ref[...].astype(jnp.bfloat16)
        r1 = [[], []]
        for b in range(B):
            qm = jnp.dot(x_ref[b].astype(jnp.bfloat16), wqb,
                         preferred_element_type=jnp.float32)
            l_cols = []
            for h in range(HQ):
                qg = _regroup(qm[:, h * DH:(h + 1) * DH]).astype(
                    jnp.bfloat16)
                kg = _regroup(k_ref[b, :, h, :]).astype(jnp.bfloat16)
                vg = _regroup(v_ref[b, :, h, :]).astype(jnp.bfloat16)
                sc = lax.dot_general(
                    qg, kg, (((2,), (2,)), ((0,), (0,))),
                    preferred_element_type=jnp.float32) * 0.125
                w = jnp.exp(sc)
                l_cols.append(
                    _ungroup(jnp.sum(w, axis=2, keepdims=True), n=1))
                og = lax.dot_general(
                    w.astype(jnp.bfloat16), vg, (((2,), (1,)), ((0,), (0,))),
                    preferred_element_type=jnp.float32)
                acc_ref[b * SQ:(b + 1) * SQ,
                        h * DH:(h + 1) * DH] = _ungroup(og)
            acc_ref[b * SQ:(b + 1) * SQ, DQK:] = jnp.concatenate(
                l_cols, axis=1)
            sendbuf_ref[b * SQ:(b + 1) * SQ, :] = acc_ref[
                b * SQ:(b + 1) * SQ, :].astype(jnp.bfloat16)

            for o in range(1, N_DEV):
                t = lax.rem(d + o, N_DEV)
                r = pltpu.make_async_remote_copy(
                    src_ref=sendbuf_ref.at[pl.ds(b * SQ + t * CHUNK, CHUNK), :],
                    dst_ref=stage_ref.at[b, o],
                    send_sem=send1.at[b, o],
                    recv_sem=recv1.at[b, o],
                    device_id=(t,),
                    device_id_type=_MESH,
                )
                r.start()
                r1[b].append(r)

        r2 = []
        for b in range(B):
            for r in r1[b]:
                r.wait()
            rows = pl.ds(b * SQ + d * CHUNK, CHUNK)
            red = acc_ref[rows, :] + jnp.sum(
                stage_ref[b, 1:, :, :].astype(jnp.float32), axis=0)
            ctx = jnp.concatenate(
                [red[:, h * DH:(h + 1) * DH] / red[:, DQK + h:DQK + h + 1]
                 for h in range(HQ)], axis=1)
            orows = pl.ds(d * CHUNK, CHUNK)
            ostage_ref[b, orows, :] = ctx.astype(jnp.bfloat16)
            for o in range(1, N_DEV):
                t = lax.rem(d + o, N_DEV)
                r = pltpu.make_async_remote_copy(
                    src_ref=ostage_ref.at[b, orows, :],
                    dst_ref=ostage_ref.at[b, orows, :],
                    send_sem=send2.at[b, o],
                    recv_sem=recv2.at[b, o],
                    device_id=(t,),
                    device_id_type=_MESH,
                )
                r.start()
                r2.append(r)
        for r in r2:
            r.wait()
        wob = wo_ref[...].astype(jnp.bfloat16)
        for b in range(B):
            out_ref[b] = jnp.dot(ostage_ref[b], wob,
                                 preferred_element_type=jnp.float32)

    return pl.pallas_call(
        body,
        out_shape=jax.ShapeDtypeStruct((B, SQ, D_MODEL), jnp.float32),
        in_specs=[pl.BlockSpec(memory_space=pltpu.VMEM)] * 5,
        out_specs=pl.BlockSpec(memory_space=pltpu.VMEM),
        scratch_shapes=[
            pltpu.VMEM((B * SQ, COLS), jnp.float32),
            pltpu.VMEM((B * SQ, COLS), jnp.bfloat16),
            pltpu.VMEM((B, N_DEV, CHUNK, COLS), jnp.bfloat16),
            pltpu.VMEM((B, SQ, DQK), jnp.bfloat16),
            pltpu.SemaphoreType.DMA((B, N_DEV)),
            pltpu.SemaphoreType.DMA((B, N_DEV)),
            pltpu.SemaphoreType.DMA((B, N_DEV)),
            pltpu.SemaphoreType.DMA((B, N_DEV)),
        ],
        compiler_params=pltpu.CompilerParams(collective_id=0),
    )(x, Wq, K_ext, V_ext, Wo)


# device time: 50059 ns/iter; 1.1182x vs baseline; 1.0440x over previous
import jax
import jax.numpy as jnp
from jax import lax
from jax.experimental import pallas as pl
from jax.experimental.pallas import tpu as pltpu

N_DEV = 32
B, SQ, D_MODEL, HQ, DH = 2, 512, 768, 8, 64
DQK = HQ * DH
SKV_LOC = 512
COLS = DQK + HQ
CHUNK = SQ // N_DEV

_MESH = pl.DeviceIdType.MESH


def _regroup(a):
    return jnp.swapaxes(a.reshape(2, 4, 64, DH), 0, 1).reshape(4, 128, DH)


def _ungroup(a, n=DH):
    return jnp.swapaxes(a.reshape(4, 2, 64, n), 0, 1).reshape(SQ, n)


def kernel(x, Wq, K_ext, V_ext, Wo):
    def body(x_ref, wq_ref, k_ref, v_ref, wo_ref, out_ref,
             acc_ref, sendbuf_ref, stage_ref, ostage_ref,
             send1, recv1, send2, recv2):
        d = lax.axis_index("i")

        barrier_sem = pltpu.get_barrier_semaphore()
        for o in range(1, N_DEV):
            t = lax.rem(d + o, N_DEV)
            pl.semaphore_signal(barrier_sem, inc=1, device_id=(t,),
                                device_id_type=_MESH)
        pl.semaphore_wait(barrier_sem, N_DEV - 1)

        r1 = [[], []]
        for b in range(B):
            qm = jnp.dot(x_ref[b], wq_ref[...],
                         preferred_element_type=jnp.float32)
            l_cols = []
            for h in range(HQ):
                qg = _regroup(qm[:, h * DH:(h + 1) * DH])
                kg = _regroup(k_ref[b, :, h, :])
                vg = _regroup(v_ref[b, :, h, :])
                sc = lax.dot_general(
                    qg, kg, (((2,), (2,)), ((0,), (0,))),
                    preferred_element_type=jnp.float32) * 0.125
                w = jnp.exp(sc)
                l_cols.append(
                    _ungroup(jnp.sum(w, axis=2, keepdims=True), n=1))
                og = lax.dot_general(
                    w, vg, (((2,), (1,)), ((0,), (0,))),
                    preferred_element_type=jnp.float32)
                acc_ref[b * SQ:(b + 1) * SQ,
                        h * DH:(h + 1) * DH] = _ungroup(og)
            acc_ref[b * SQ:(b + 1) * SQ, DQK:] = jnp.concatenate(
                l_cols, axis=1)
            sendbuf_ref[b * SQ:(b + 1) * SQ, :] = acc_ref[
                b * SQ:(b + 1) * SQ, :].astype(jnp.bfloat16)

            for o in range(1, N_DEV):
                t = lax.rem(d + o, N_DEV)
                r = pltpu.make_async_remote_copy(
                    src_ref=sendbuf_ref.at[pl.ds(b * SQ + t * CHUNK, CHUNK), :],
                    dst_ref=stage_ref.at[b, o],
                    send_sem=send1.at[b, o],
                    recv_sem=recv1.at[b, o],
                    device_id=(t,),
                    device_id_type=_MESH,
                )
                r.start()
                r1[b].append(r)

        r2 = [[], []]
        for b in range(B):
            for r in r1[b]:
                r.wait()
            rows = pl.ds(b * SQ + d * CHUNK, CHUNK)
            red = acc_ref[rows, :] + jnp.sum(
                stage_ref[b, 1:, :, :].astype(jnp.float32), axis=0)
            ctx = jnp.concatenate(
                [red[:, h * DH:(h + 1) * DH] / red[:, DQK + h:DQK + h + 1]
                 for h in range(HQ)], axis=1)
            orows = pl.ds(d * CHUNK, CHUNK)
            ostage_ref[b, orows, :] = ctx.astype(jnp.bfloat16)
            for o in range(1, N_DEV):
                t = lax.rem(d + o, N_DEV)
                r = pltpu.make_async_remote_copy(
                    src_ref=ostage_ref.at[b, orows, :],
                    dst_ref=ostage_ref.at[b, orows, :],
                    send_sem=send2.at[b, o],
                    recv_sem=recv2.at[b, o],
                    device_id=(t,),
                    device_id_type=_MESH,
                )
                r.start()
                r2[b].append(r)
        wob = wo_ref[...].astype(jnp.bfloat16)
        for b in range(B):
            for r in r2[b]:
                r.wait()
            out_ref[b] = jnp.dot(ostage_ref[b], wob,
                                 preferred_element_type=jnp.float32)

    return pl.pallas_call(
        body,
        out_shape=jax.ShapeDtypeStruct((B, SQ, D_MODEL), jnp.float32),
        in_specs=[pl.BlockSpec(memory_space=pltpu.VMEM)] * 5,
        out_specs=pl.BlockSpec(memory_space=pltpu.VMEM),
        scratch_shapes=[
            pltpu.VMEM((B * SQ, COLS), jnp.float32),
            pltpu.VMEM((B * SQ, COLS), jnp.bfloat16),
            pltpu.VMEM((B, N_DEV, CHUNK, COLS), jnp.bfloat16),
            pltpu.VMEM((B, SQ, DQK), jnp.bfloat16),
            pltpu.SemaphoreType.DMA((B, N_DEV)),
            pltpu.SemaphoreType.DMA((B, N_DEV)),
            pltpu.SemaphoreType.DMA((B, N_DEV)),
            pltpu.SemaphoreType.DMA((B, N_DEV)),
        ],
        compiler_params=pltpu.CompilerParams(collective_id=0),
    )(x, Wq, K_ext, V_ext, Wo)
